# Initial kernel scaffold; baseline (speedup 1.0000x reference)
import jax
import jax.numpy as jnp
from jax.experimental import pallas as pl


def kernel(x, W, codebook):
    raise NotImplementedError("write your pallas kernel here")


# speedup vs baseline: 6.6961x; 6.6961x over previous
import functools

import jax
import jax.numpy as jnp
from jax.experimental import pallas as pl

DIM = 768
CODEBOOK_SIZE = 1024
CODEBOOK_DIM = 32
EPS = 1e-5

BLK = 512


def _rpq_kernel(x_ref, w_ref, cb_ref, out_ref):
    proj = jax.lax.dot_general(
        x_ref[...], w_ref[...],
        dimension_numbers=(((1,), (1,)), ((), ())),
        preferred_element_type=jnp.float32,
        precision=jax.lax.Precision.DEFAULT,
    )
    mean = jnp.mean(proj, axis=-1, keepdims=True)
    var = jnp.mean((proj - mean) ** 2, axis=-1, keepdims=True)
    p = (proj - mean) / jnp.sqrt(var + EPS)

    cb = cb_ref[...]
    cn = jax.lax.dot_general(
        jnp.ones((1, CODEBOOK_DIM), jnp.float32), cb * cb,
        dimension_numbers=(((1,), (1,)), ((), ())),
        preferred_element_type=jnp.float32,
        precision=jax.lax.Precision.HIGHEST,
    )
    s = jax.lax.dot_general(
        p, cb,
        dimension_numbers=(((1,), (1,)), ((), ())),
        preferred_element_type=jnp.float32,
        precision=jax.lax.Precision.HIGHEST,
    )
    d = cn - 2.0 * s
    out_ref[...] = jnp.argmin(d, axis=-1).astype(jnp.int32)


@jax.jit
def kernel(x, W, codebook):
    B, L, _ = x.shape
    n_tok = B * L
    xf = x.reshape(n_tok, DIM)
    grid = n_tok // BLK
    codes = pl.pallas_call(
        _rpq_kernel,
        grid=(grid,),
        in_specs=[
            pl.BlockSpec((BLK, DIM), lambda i: (i, 0)),
            pl.BlockSpec((CODEBOOK_DIM, DIM), lambda i: (0, 0)),
            pl.BlockSpec((CODEBOOK_SIZE, CODEBOOK_DIM), lambda i: (0, 0)),
        ],
        out_specs=pl.BlockSpec((BLK,), lambda i: (i,)),
        out_shape=jax.ShapeDtypeStruct((n_tok,), jnp.int32),
    )(xf, W, codebook)
    return codes.reshape(B, L)


# speedup vs baseline: 7.2637x; 1.0848x over previous
import functools

import jax
import jax.numpy as jnp
from jax.experimental import pallas as pl

DIM = 768
CODEBOOK_SIZE = 1024
CODEBOOK_DIM = 32
EPS = 1e-5

BLK = 512


def _rpq_kernel(x_ref, w_ref, cb_ref, out_ref):
    proj = jax.lax.dot_general(
        x_ref[...], w_ref[...],
        dimension_numbers=(((1,), (1,)), ((), ())),
        preferred_element_type=jnp.float32,
        precision=jax.lax.Precision.DEFAULT,
    )
    mean = jnp.mean(proj, axis=-1, keepdims=True)
    var = jnp.mean((proj - mean) ** 2, axis=-1, keepdims=True)
    p = (proj - mean) / jnp.sqrt(var + EPS)

    cb = cb_ref[...]
    cn = jnp.sum(cb * cb, axis=1, keepdims=True)
    b_aug = jnp.concatenate([-2.0 * cb, cn], axis=1)
    a_aug = jnp.concatenate([p, jnp.ones((p.shape[0], 1), jnp.float32)], axis=1)
    d = jax.lax.dot_general(
        a_aug, b_aug,
        dimension_numbers=(((1,), (1,)), ((), ())),
        preferred_element_type=jnp.float32,
        precision=jax.lax.Precision.HIGHEST,
    )
    out_ref[...] = jnp.argmin(d, axis=-1).astype(jnp.int32)


@jax.jit
def kernel(x, W, codebook):
    B, L, _ = x.shape
    n_tok = B * L
    xf = x.reshape(n_tok, DIM)
    grid = n_tok // BLK
    codes = pl.pallas_call(
        _rpq_kernel,
        grid=(grid,),
        in_specs=[
            pl.BlockSpec((BLK, DIM), lambda i: (i, 0)),
            pl.BlockSpec((CODEBOOK_DIM, DIM), lambda i: (0, 0)),
            pl.BlockSpec((CODEBOOK_SIZE, CODEBOOK_DIM), lambda i: (0, 0)),
        ],
        out_specs=pl.BlockSpec((BLK,), lambda i: (i,)),
        out_shape=jax.ShapeDtypeStruct((n_tok,), jnp.int32),
    )(xf, W, codebook)
    return codes.reshape(B, L)


# speedup vs baseline: 7.6085x; 1.0475x over previous
import functools

import jax
import jax.numpy as jnp
from jax.experimental import pallas as pl

DIM = 768
CODEBOOK_SIZE = 1024
CODEBOOK_DIM = 32
EPS = 1e-5

BLK = 1024


def _rpq_kernel(x_ref, w_ref, cb_ref, out_ref):
    proj = jax.lax.dot_general(
        x_ref[...], w_ref[...],
        dimension_numbers=(((1,), (1,)), ((), ())),
        preferred_element_type=jnp.float32,
        precision=jax.lax.Precision.DEFAULT,
    )
    mean = jnp.mean(proj, axis=-1, keepdims=True)
    var = jnp.mean((proj - mean) ** 2, axis=-1, keepdims=True)
    p = (proj - mean) / jnp.sqrt(var + EPS)

    cb = cb_ref[...]
    cn = jnp.sum(cb * cb, axis=1, keepdims=True)
    b_aug = jnp.concatenate([-2.0 * cb, cn], axis=1)
    a_aug = jnp.concatenate([p, jnp.ones((p.shape[0], 1), jnp.float32)], axis=1)
    d = jax.lax.dot_general(
        a_aug, b_aug,
        dimension_numbers=(((1,), (1,)), ((), ())),
        preferred_element_type=jnp.float32,
        precision=jax.lax.Precision.HIGHEST,
    )
    out_ref[...] = jnp.argmin(d, axis=-1).astype(jnp.int32)


@jax.jit
def kernel(x, W, codebook):
    B, L, _ = x.shape
    n_tok = B * L
    xf = x.reshape(n_tok, DIM)
    grid = n_tok // BLK
    codes = pl.pallas_call(
        _rpq_kernel,
        grid=(grid,),
        in_specs=[
            pl.BlockSpec((BLK, DIM), lambda i: (i, 0)),
            pl.BlockSpec((CODEBOOK_DIM, DIM), lambda i: (0, 0)),
            pl.BlockSpec((CODEBOOK_SIZE, CODEBOOK_DIM), lambda i: (0, 0)),
        ],
        out_specs=pl.BlockSpec((BLK,), lambda i: (i,)),
        out_shape=jax.ShapeDtypeStruct((n_tok,), jnp.int32),
    )(xf, W, codebook)
    return codes.reshape(B, L)
